# baseline (speedup 1.0000x reference)
import functools

import jax
import jax.numpy as jnp
from jax import lax
from jax.experimental import pallas as pl
from jax.experimental.pallas import tpu as pltpu
from jax.experimental.pallas import tpu_sc as plsc

N = 10000
E = 320000
IN_DIM = 128
HID = 128
OUT = 64
HALF = 32
K = 10
ALPHA = 0.1

NPAD = 10240
RT = NPAD // 16
CH = 160
EPT = CH * 128
E_PAD = 16 * EPT
DEG_CH = E_PAD // (32 * 128)

_f32 = jnp.float32


def _mlp_body(feats_ref, w1_ref, b1_ref, w2_ref, b2_ref, h1_ref, h_ref):
    a = jnp.dot(feats_ref[...], w1_ref[...],
                preferred_element_type=_f32) + b1_ref[...]
    h1_ref[...] = a
    h_ref[...] = jnp.dot(jnp.maximum(a, 0.0), w2_ref[...],
                         preferred_element_type=_f32) + b2_ref[...]


def _mlp(feats, W1, b1, W2, b2):
    RB = 2000
    grid = (N // RB,)
    return pl.pallas_call(
        _mlp_body,
        grid=grid,
        in_specs=[
            pl.BlockSpec((RB, IN_DIM), lambda i: (i, 0)),
            pl.BlockSpec((IN_DIM, HID), lambda i: (0, 0)),
            pl.BlockSpec((1, HID), lambda i: (0, 0)),
            pl.BlockSpec((HID, OUT), lambda i: (0, 0)),
            pl.BlockSpec((1, OUT), lambda i: (0, 0)),
        ],
        out_specs=[
            pl.BlockSpec((RB, HID), lambda i: (i, 0)),
            pl.BlockSpec((RB, OUT), lambda i: (i, 0)),
        ],
        out_shape=[
            jax.ShapeDtypeStruct((N, HID), _f32),
            jax.ShapeDtypeStruct((N, OUT), _f32),
        ],
    )(feats, W1.astype(_f32), b1.reshape(1, HID), W2.astype(_f32),
      b2.reshape(1, OUT))


def _deg_body(dst_hbm, deg_out, idx_v, ones_v, zb_v, deg_sh):
    c = lax.axis_index("core")
    s = lax.axis_index("subcore")
    wid = c * 16 + s

    @pl.loop(0, 8)
    def _(i):
        ones_v[pl.ds(i * 16, 16)] = jnp.full((16,), 1.0, _f32)
        zb_v[pl.ds(i * 16, 16)] = jnp.zeros((16,), _f32)

    @pl.loop(0, 5)
    def _(z):
        pltpu.sync_copy(zb_v, deg_sh.at[pl.ds(s * RT + z * 128, 128)])

    pltpu.sync_copy(dst_hbm.at[wid], idx_v)
    plsc.subcore_barrier()

    @pl.loop(0, DEG_CH)
    def _(j):
        pltpu.sync_copy(ones_v, deg_sh.at[idx_v.at[j]], add=True)

    plsc.subcore_barrier()

    @pl.when(s == 0)
    def _():
        pltpu.sync_copy(deg_sh, deg_out.at[c])


def _degree(dst_deg):
    mesh = plsc.VectorSubcoreMesh(core_axis_name="core",
                                  subcore_axis_name="subcore")
    f = pl.kernel(
        _deg_body,
        out_type=jax.ShapeDtypeStruct((2, NPAD), _f32),
        mesh=mesh,
        scratch_types=[
            pltpu.VMEM((DEG_CH, 128), jnp.int32),
            pltpu.VMEM((128,), _f32),
            pltpu.VMEM((128,), _f32),
            pltpu.VMEM_SHARED((NPAD,), _f32),
        ],
    )
    return f(dst_deg)


def _prep_body(deg_ref, h_ref, c1_ref, g0_ref, c2_ref, invx_ref):
    i = pl.program_id(0)
    rb = deg_ref.shape[0]
    rows = i * rb + lax.broadcasted_iota(jnp.int32, (rb, 1), 0)
    valid = rows < N
    d = jnp.maximum(deg_ref[...], 1.0)
    norm = jnp.where(valid, lax.rsqrt(d), 0.0)
    inv = jnp.where(valid, jnp.sqrt(d), 0.0)
    h = h_ref[...]
    g0lo = norm * h[:, :HALF]
    g0hi = norm * h[:, HALF:]
    g0_ref[0] = g0lo
    g0_ref[1] = g0hi
    c2_ref[0] = ALPHA * g0lo
    c2_ref[1] = ALPHA * g0hi
    ones = jnp.ones((1, HALF), _f32)
    c1_ref[...] = ((1.0 - ALPHA) * norm * norm) * ones
    invx_ref[0] = inv * ones
    invx_ref[1] = inv * ones


def _prep(deg_col, h_pad):
    RB = 2560
    grid = (NPAD // RB,)
    return pl.pallas_call(
        _prep_body,
        grid=grid,
        in_specs=[
            pl.BlockSpec((RB, 1), lambda i: (i, 0)),
            pl.BlockSpec((RB, OUT), lambda i: (i, 0)),
        ],
        out_specs=[
            pl.BlockSpec((RB, HALF), lambda i: (i, 0)),
            pl.BlockSpec((2, RB, HALF), lambda i: (0, i, 0)),
            pl.BlockSpec((2, RB, HALF), lambda i: (0, i, 0)),
            pl.BlockSpec((2, RB, HALF), lambda i: (0, i, 0)),
        ],
        out_shape=[
            jax.ShapeDtypeStruct((NPAD, HALF), _f32),
            jax.ShapeDtypeStruct((2, NPAD, HALF), _f32),
            jax.ShapeDtypeStruct((2, NPAD, HALF), _f32),
            jax.ShapeDtypeStruct((2, NPAD, HALF), _f32),
        ],
    )(deg_col, h_pad)


BLK = 32
NBLK = CH // BLK


def _prop_body(src_hbm, dst_hbm, c1_hbm, c2_hbm, g0_hbm, gk_out,
               sidx, didx, rows, c1_b, c2_b, nbuf, zbuf, gs, ss, agg_sh):
    c = lax.axis_index("core")
    s = lax.axis_index("subcore")
    rowbase = s * RT
    hrowbase = c * NPAD + rowbase
    schunkbase = c * 16 * CH + s * CH
    dchunkbase = s * CH

    pltpu.sync_copy(c1_hbm.at[pl.ds(rowbase, RT)], c1_b)
    pltpu.sync_copy(c2_hbm.at[pl.ds(hrowbase, RT)], c2_b)
    pltpu.sync_copy(g0_hbm.at[pl.ds(hrowbase, RT)], nbuf)
    pltpu.sync_copy(nbuf, gk_out.at[pl.ds(hrowbase, RT)])

    @pl.loop(0, 64)
    def _(i):
        zbuf[i, pl.ds(0, 16)] = jnp.zeros((16,), _f32)
        zbuf[i, pl.ds(16, 16)] = jnp.zeros((16,), _f32)

    @pl.loop(0, 10)
    def _(z):
        pltpu.sync_copy(zbuf, agg_sh.at[pl.ds(rowbase + z * 64, 64)])

    plsc.subcore_barrier()

    def wait_gather(j, m):
        pltpu.make_async_copy(gk_out.at[sidx.at[j]], rows.at[m],
                              gs.at[m]).wait()

    def wait_scatter(j, m):
        pltpu.make_async_copy(rows.at[m], agg_sh.at[didx.at[j]],
                              ss.at[m]).wait()

    def fire_gather(j, m):
        pltpu.async_copy(gk_out.at[sidx.at[j]], rows.at[m], gs.at[m])

    def fire_scatter(j, m):
        pltpu.async_copy(rows.at[m], agg_sh.at[didx.at[j]], ss.at[m],
                         add=True)

    @pl.loop(0, K)
    def _(t):
        @pl.loop(0, NBLK)
        def _(b):
            pltpu.sync_copy(src_hbm.at[pl.ds(schunkbase + b * BLK, BLK)], sidx)
            pltpu.sync_copy(dst_hbm.at[pl.ds(dchunkbase + b * BLK, BLK)], didx)

            @pl.loop(0, BLK, step=4)
            def _(j0):
                for cc in range(4):
                    j = j0 + cc
                    m = cc
                    m2 = (cc + 2) % 4

                    @pl.when(j >= 4)
                    def _():
                        wait_scatter(j - 4, m)

                    fire_gather(j, m)

                    @pl.when(j >= 2)
                    def _():
                        wait_gather(j - 2, m2)
                        fire_scatter(j - 2, m2)

            for j in (BLK - 2, BLK - 1):
                m = j % 4
                wait_gather(j, m)
                fire_scatter(j, m)
            for j in (BLK - 4, BLK - 3, BLK - 2, BLK - 1):
                wait_scatter(j, j % 4)

        plsc.subcore_barrier()

        pltpu.sync_copy(agg_sh.at[pl.ds(rowbase, RT)], nbuf)

        @pl.loop(0, RT, unroll=8)
        def _(i):
            lo = pl.ds(0, 16)
            hi = pl.ds(16, 16)
            nbuf[i, lo] = c1_b[i, lo] * nbuf[i, lo] + c2_b[i, lo]
            nbuf[i, hi] = c1_b[i, hi] * nbuf[i, hi] + c2_b[i, hi]

        pltpu.sync_copy(nbuf, gk_out.at[pl.ds(hrowbase, RT)])

        @pl.loop(0, 10)
        def _(z):
            pltpu.sync_copy(zbuf, agg_sh.at[pl.ds(rowbase + z * 64, 64)])

        plsc.subcore_barrier()


def _propagate(src_idx, dst_idx, c1x, c2f, g0f):
    mesh = plsc.VectorSubcoreMesh(core_axis_name="core",
                                  subcore_axis_name="subcore")
    f = pl.kernel(
        _prop_body,
        out_type=jax.ShapeDtypeStruct((2 * NPAD, HALF), _f32),
        mesh=mesh,
        compiler_params=pltpu.CompilerParams(use_tc_tiling_on_sc=False),
        scratch_types=[
            pltpu.VMEM((BLK, 128), jnp.int32),
            pltpu.VMEM((BLK, 128), jnp.int32),
            pltpu.VMEM((4, 128, HALF), _f32),
            pltpu.VMEM((RT, HALF), _f32),
            pltpu.VMEM((RT, HALF), _f32),
            pltpu.VMEM((RT, HALF), _f32),
            pltpu.VMEM((64, HALF), _f32),
            pltpu.SemaphoreType.DMA((4,)),
            pltpu.SemaphoreType.DMA((4,)),
            pltpu.VMEM_SHARED((NPAD, HALF), _f32),
        ],
    )
    return f(src_idx, dst_idx, c1x, c2f, g0f)


def _final_body(gk_ref, invx_ref, h_ref):
    h_ref[...] = jnp.concatenate(
        [gk_ref[0] * invx_ref[0], gk_ref[1] * invx_ref[1]], axis=1)


def _final(gk, invx):
    RB = 2000
    grid = (N // RB,)
    return pl.pallas_call(
        _final_body,
        grid=grid,
        in_specs=[
            pl.BlockSpec((2, RB, HALF), lambda i: (0, i, 0)),
            pl.BlockSpec((2, RB, HALF), lambda i: (0, i, 0)),
        ],
        out_specs=pl.BlockSpec((RB, OUT), lambda i: (i, 0)),
        out_shape=jax.ShapeDtypeStruct((N, OUT), _f32),
    )(gk, invx)


@jax.jit
def kernel(feats, edge_index, W1, b1, W2, b2):
    src = edge_index[0]
    dst = edge_index[1]
    pad = E_PAD - E
    src_p = jnp.concatenate([src, jnp.full((pad,), N, jnp.int32)])
    dst_p = jnp.concatenate([dst, jnp.full((pad,), N, jnp.int32)])
    src_idx = jnp.concatenate([src_p, src_p + NPAD]).reshape(2 * 16 * CH, 128)
    dst_idx = dst_p.reshape(16 * CH, 128)
    dst_deg = dst_p.reshape(32, DEG_CH, 128)

    h1, h = _mlp(feats, W1, b1, W2, b2)
    deg2 = _degree(dst_deg)
    deg_col = (deg2[0] + deg2[1]).reshape(NPAD, 1)
    h_pad = jnp.pad(h, ((0, NPAD - N), (0, 0)))
    c1x, g0, c2, invx = _prep(deg_col, h_pad)
    gk = _propagate(src_idx, dst_idx, c1x,
                    c2.reshape(2 * NPAD, HALF), g0.reshape(2 * NPAD, HALF))
    hout = _final(gk.reshape(2, NPAD, HALF), invx)
    return (h1, hout)


# speedup vs baseline: 1.6787x; 1.6787x over previous
import functools

import jax
import jax.numpy as jnp
from jax import lax
from jax.experimental import pallas as pl
from jax.experimental.pallas import tpu as pltpu
from jax.experimental.pallas import tpu_sc as plsc

N = 10000
E = 320000
IN_DIM = 128
HID = 128
OUT = 64
HALF = 32
K = 10
ALPHA = 0.1

NPAD = 10240
RT = NPAD // 16
CH = 160
EPT = CH * 128
E_PAD = 16 * EPT
DEG_CH = E_PAD // (32 * 128)

_f32 = jnp.float32


def _mlp_body(feats_ref, w1_ref, b1_ref, w2_ref, b2_ref, h1_ref, h_ref):
    a = jnp.dot(feats_ref[...], w1_ref[...],
                preferred_element_type=_f32) + b1_ref[...]
    h1_ref[...] = a
    h_ref[...] = jnp.dot(jnp.maximum(a, 0.0), w2_ref[...],
                         preferred_element_type=_f32) + b2_ref[...]


def _mlp(feats, W1, b1, W2, b2):
    RB = 2000
    grid = (N // RB,)
    return pl.pallas_call(
        _mlp_body,
        grid=grid,
        in_specs=[
            pl.BlockSpec((RB, IN_DIM), lambda i: (i, 0)),
            pl.BlockSpec((IN_DIM, HID), lambda i: (0, 0)),
            pl.BlockSpec((1, HID), lambda i: (0, 0)),
            pl.BlockSpec((HID, OUT), lambda i: (0, 0)),
            pl.BlockSpec((1, OUT), lambda i: (0, 0)),
        ],
        out_specs=[
            pl.BlockSpec((RB, HID), lambda i: (i, 0)),
            pl.BlockSpec((RB, OUT), lambda i: (i, 0)),
        ],
        out_shape=[
            jax.ShapeDtypeStruct((N, HID), _f32),
            jax.ShapeDtypeStruct((N, OUT), _f32),
        ],
    )(feats, W1.astype(_f32), b1.reshape(1, HID), W2.astype(_f32),
      b2.reshape(1, OUT))


def _deg_body(dst_hbm, deg_out, idx_v, ones_v, zb_v, deg_sh):
    c = lax.axis_index("core")
    s = lax.axis_index("subcore")
    wid = c * 16 + s

    @pl.loop(0, 8)
    def _(i):
        ones_v[pl.ds(i * 16, 16)] = jnp.full((16,), 1.0, _f32)
        zb_v[pl.ds(i * 16, 16)] = jnp.zeros((16,), _f32)

    @pl.loop(0, 5)
    def _(z):
        pltpu.sync_copy(zb_v, deg_sh.at[pl.ds(s * RT + z * 128, 128)])

    pltpu.sync_copy(dst_hbm.at[wid], idx_v)
    plsc.subcore_barrier()

    @pl.loop(0, DEG_CH)
    def _(j):
        pltpu.sync_copy(ones_v, deg_sh.at[idx_v.at[j]], add=True)

    plsc.subcore_barrier()

    @pl.when(s == 0)
    def _():
        pltpu.sync_copy(deg_sh, deg_out.at[c])


def _degree(dst_deg):
    mesh = plsc.VectorSubcoreMesh(core_axis_name="core",
                                  subcore_axis_name="subcore")
    f = pl.kernel(
        _deg_body,
        out_type=jax.ShapeDtypeStruct((2, NPAD), _f32),
        mesh=mesh,
        scratch_types=[
            pltpu.VMEM((DEG_CH, 128), jnp.int32),
            pltpu.VMEM((128,), _f32),
            pltpu.VMEM((128,), _f32),
            pltpu.VMEM_SHARED((NPAD,), _f32),
        ],
    )
    return f(dst_deg)


def _prep_body(deg_ref, h_ref, c1_ref, g0_ref, c2_ref, invx_ref):
    i = pl.program_id(0)
    rb = deg_ref.shape[0]
    rows = i * rb + lax.broadcasted_iota(jnp.int32, (rb, 1), 0)
    valid = rows < N
    d = jnp.maximum(deg_ref[...], 1.0)
    norm = jnp.where(valid, lax.rsqrt(d), 0.0)
    inv = jnp.where(valid, jnp.sqrt(d), 0.0)
    h = h_ref[...]
    g0lo = norm * h[:, :HALF]
    g0hi = norm * h[:, HALF:]
    g0_ref[0] = g0lo
    g0_ref[1] = g0hi
    c2_ref[0] = ALPHA * g0lo
    c2_ref[1] = ALPHA * g0hi
    ones = jnp.ones((1, HALF), _f32)
    c1_ref[...] = ((1.0 - ALPHA) * norm * norm) * ones
    invx_ref[0] = inv * ones
    invx_ref[1] = inv * ones


def _prep(deg_col, h_pad):
    RB = 2560
    grid = (NPAD // RB,)
    return pl.pallas_call(
        _prep_body,
        grid=grid,
        in_specs=[
            pl.BlockSpec((RB, 1), lambda i: (i, 0)),
            pl.BlockSpec((RB, OUT), lambda i: (i, 0)),
        ],
        out_specs=[
            pl.BlockSpec((RB, HALF), lambda i: (i, 0)),
            pl.BlockSpec((2, RB, HALF), lambda i: (0, i, 0)),
            pl.BlockSpec((2, RB, HALF), lambda i: (0, i, 0)),
            pl.BlockSpec((2, RB, HALF), lambda i: (0, i, 0)),
        ],
        out_shape=[
            jax.ShapeDtypeStruct((NPAD, HALF), _f32),
            jax.ShapeDtypeStruct((2, NPAD, HALF), _f32),
            jax.ShapeDtypeStruct((2, NPAD, HALF), _f32),
            jax.ShapeDtypeStruct((2, NPAD, HALF), _f32),
        ],
    )(deg_col, h_pad)


CHUNK = 256
CCH = EPT // CHUNK
BLK = 16
NBLK = CCH // BLK


def _prop_body(src_hbm, dst_hbm, c1_hbm, c2_hbm, g0_hbm, gk_out,
               sidx, didx, rows, c1_b, c2_b, nbuf, zbuf, gs, ss, g_sh, agg_sh):
    c = lax.axis_index("core")
    s = lax.axis_index("subcore")
    rowbase = s * RT
    hrowbase = c * NPAD + rowbase
    chunkbase = s * CCH

    pltpu.sync_copy(c1_hbm.at[pl.ds(rowbase, RT)], c1_b)
    pltpu.sync_copy(c2_hbm.at[pl.ds(hrowbase, RT)], c2_b)
    pltpu.sync_copy(g0_hbm.at[pl.ds(hrowbase, RT)], g_sh.at[pl.ds(rowbase, RT)])

    @pl.loop(0, 64)
    def _(i):
        zbuf[i, pl.ds(0, 16)] = jnp.zeros((16,), _f32)
        zbuf[i, pl.ds(16, 16)] = jnp.zeros((16,), _f32)

    @pl.loop(0, 10)
    def _(z):
        pltpu.sync_copy(zbuf, agg_sh.at[pl.ds(rowbase + z * 64, 64)])

    plsc.subcore_barrier()

    def wait_gather(j, m):
        pltpu.make_async_copy(g_sh.at[sidx.at[j]], rows.at[m],
                              gs.at[m]).wait()

    def wait_scatter(j, m):
        pltpu.make_async_copy(rows.at[m], agg_sh.at[didx.at[j]],
                              ss.at[m]).wait()

    def fire_gather(j, m):
        pltpu.async_copy(g_sh.at[sidx.at[j]], rows.at[m], gs.at[m])

    def fire_scatter(j, m):
        pltpu.async_copy(rows.at[m], agg_sh.at[didx.at[j]], ss.at[m],
                         add=True)

    @pl.loop(0, K)
    def _(t):
        @pl.loop(0, NBLK)
        def _(b):
            pltpu.sync_copy(src_hbm.at[pl.ds(chunkbase + b * BLK, BLK)], sidx)
            pltpu.sync_copy(dst_hbm.at[pl.ds(chunkbase + b * BLK, BLK)], didx)

            @pl.loop(0, BLK, step=2)
            def _(j0):
                for cc in range(2):
                    j = j0 + cc
                    m = cc
                    m1 = (cc + 1) % 2

                    @pl.when(j >= 2)
                    def _():
                        wait_scatter(j - 2, m)

                    fire_gather(j, m)

                    @pl.when(j >= 1)
                    def _():
                        wait_gather(j - 1, m1)
                        fire_scatter(j - 1, m1)

            j = BLK - 1
            wait_gather(j, j % 2)
            fire_scatter(j, j % 2)
            wait_scatter(BLK - 2, 0)
            wait_scatter(BLK - 1, 1)

        plsc.subcore_barrier()

        pltpu.sync_copy(agg_sh.at[pl.ds(rowbase, RT)], nbuf)

        @pl.loop(0, RT, unroll=8)
        def _(i):
            lo = pl.ds(0, 16)
            hi = pl.ds(16, 16)
            nbuf[i, lo] = c1_b[i, lo] * nbuf[i, lo] + c2_b[i, lo]
            nbuf[i, hi] = c1_b[i, hi] * nbuf[i, hi] + c2_b[i, hi]

        pltpu.sync_copy(nbuf, g_sh.at[pl.ds(rowbase, RT)])

        @pl.loop(0, 10)
        def _(z):
            pltpu.sync_copy(zbuf, agg_sh.at[pl.ds(rowbase + z * 64, 64)])

        plsc.subcore_barrier()

    pltpu.sync_copy(nbuf, gk_out.at[pl.ds(hrowbase, RT)])


def _propagate(src_idx, dst_idx, c1x, c2f, g0f):
    mesh = plsc.VectorSubcoreMesh(core_axis_name="core",
                                  subcore_axis_name="subcore")
    f = pl.kernel(
        _prop_body,
        out_type=jax.ShapeDtypeStruct((2 * NPAD, HALF), _f32),
        mesh=mesh,
        compiler_params=pltpu.CompilerParams(use_tc_tiling_on_sc=False),
        scratch_types=[
            pltpu.VMEM((BLK, CHUNK), jnp.int32),
            pltpu.VMEM((BLK, CHUNK), jnp.int32),
            pltpu.VMEM((2, CHUNK, HALF), _f32),
            pltpu.VMEM((RT, HALF), _f32),
            pltpu.VMEM((RT, HALF), _f32),
            pltpu.VMEM((RT, HALF), _f32),
            pltpu.VMEM((64, HALF), _f32),
            pltpu.SemaphoreType.DMA((2,)),
            pltpu.SemaphoreType.DMA((2,)),
            pltpu.VMEM_SHARED((NPAD, HALF), _f32),
            pltpu.VMEM_SHARED((NPAD, HALF), _f32),
        ],
    )
    return f(src_idx, dst_idx, c1x, c2f, g0f)


def _final_body(gk_ref, invx_ref, h_ref):
    h_ref[...] = jnp.concatenate(
        [gk_ref[0] * invx_ref[0], gk_ref[1] * invx_ref[1]], axis=1)


def _final(gk, invx):
    RB = 2000
    grid = (N // RB,)
    return pl.pallas_call(
        _final_body,
        grid=grid,
        in_specs=[
            pl.BlockSpec((2, RB, HALF), lambda i: (0, i, 0)),
            pl.BlockSpec((2, RB, HALF), lambda i: (0, i, 0)),
        ],
        out_specs=pl.BlockSpec((RB, OUT), lambda i: (i, 0)),
        out_shape=jax.ShapeDtypeStruct((N, OUT), _f32),
    )(gk, invx)


@jax.jit
def kernel(feats, edge_index, W1, b1, W2, b2):
    src = edge_index[0]
    dst = edge_index[1]
    pad = E_PAD - E
    src_p = jnp.concatenate([src, jnp.full((pad,), N, jnp.int32)])
    dst_p = jnp.concatenate([dst, jnp.full((pad,), N, jnp.int32)])
    src_idx = src_p.reshape(16 * CCH, CHUNK)
    dst_idx = dst_p.reshape(16 * CCH, CHUNK)
    dst_deg = dst_p.reshape(32, DEG_CH, 128)

    h1, h = _mlp(feats, W1, b1, W2, b2)
    deg2 = _degree(dst_deg)
    deg_col = (deg2[0] + deg2[1]).reshape(NPAD, 1)
    h_pad = jnp.pad(h, ((0, NPAD - N), (0, 0)))
    c1x, g0, c2, invx = _prep(deg_col, h_pad)
    gk = _propagate(src_idx, dst_idx, c1x,
                    c2.reshape(2 * NPAD, HALF), g0.reshape(2 * NPAD, HALF))
    hout = _final(gk.reshape(2, NPAD, HALF), invx)
    return (h1, hout)


# speedup vs baseline: 1.8630x; 1.1098x over previous
import functools

import jax
import jax.numpy as jnp
from jax import lax
from jax.experimental import pallas as pl
from jax.experimental.pallas import tpu as pltpu
from jax.experimental.pallas import tpu_sc as plsc

N = 10000
E = 320000
IN_DIM = 128
HID = 128
OUT = 64
HALF = 32
K = 10
ALPHA = 0.1

NPAD = 10240
RT = NPAD // 16
CH = 160
EPT = CH * 128
E_PAD = 16 * EPT
DEG_CH = E_PAD // (32 * 128)

_f32 = jnp.float32


def _mlp_body(feats_ref, w1_ref, b1_ref, w2_ref, b2_ref, h1_ref, h_ref):
    a = jnp.dot(feats_ref[...], w1_ref[...],
                preferred_element_type=_f32) + b1_ref[...]
    h1_ref[...] = a
    h_ref[...] = jnp.dot(jnp.maximum(a, 0.0), w2_ref[...],
                         preferred_element_type=_f32) + b2_ref[...]


def _mlp(feats, W1, b1, W2, b2):
    RB = 2000
    grid = (N // RB,)
    return pl.pallas_call(
        _mlp_body,
        grid=grid,
        in_specs=[
            pl.BlockSpec((RB, IN_DIM), lambda i: (i, 0)),
            pl.BlockSpec((IN_DIM, HID), lambda i: (0, 0)),
            pl.BlockSpec((1, HID), lambda i: (0, 0)),
            pl.BlockSpec((HID, OUT), lambda i: (0, 0)),
            pl.BlockSpec((1, OUT), lambda i: (0, 0)),
        ],
        out_specs=[
            pl.BlockSpec((RB, HID), lambda i: (i, 0)),
            pl.BlockSpec((RB, OUT), lambda i: (i, 0)),
        ],
        out_shape=[
            jax.ShapeDtypeStruct((N, HID), _f32),
            jax.ShapeDtypeStruct((N, OUT), _f32),
        ],
    )(feats, W1.astype(_f32), b1.reshape(1, HID), W2.astype(_f32),
      b2.reshape(1, OUT))


def _deg_body(dst_hbm, deg_out, idx_v, ones_v, zb_v, deg_sh):
    c = lax.axis_index("core")
    s = lax.axis_index("subcore")
    wid = c * 16 + s

    @pl.loop(0, 8)
    def _(i):
        ones_v[pl.ds(i * 16, 16)] = jnp.full((16,), 1.0, _f32)
        zb_v[pl.ds(i * 16, 16)] = jnp.zeros((16,), _f32)

    @pl.loop(0, 5)
    def _(z):
        pltpu.sync_copy(zb_v, deg_sh.at[pl.ds(s * RT + z * 128, 128)])

    pltpu.sync_copy(dst_hbm.at[wid], idx_v)
    plsc.subcore_barrier()

    @pl.loop(0, DEG_CH)
    def _(j):
        pltpu.sync_copy(ones_v, deg_sh.at[idx_v.at[j]], add=True)

    plsc.subcore_barrier()

    @pl.when(s == 0)
    def _():
        pltpu.sync_copy(deg_sh, deg_out.at[c])


def _degree(dst_deg):
    mesh = plsc.VectorSubcoreMesh(core_axis_name="core",
                                  subcore_axis_name="subcore")
    f = pl.kernel(
        _deg_body,
        out_type=jax.ShapeDtypeStruct((2, NPAD), _f32),
        mesh=mesh,
        scratch_types=[
            pltpu.VMEM((DEG_CH, 128), jnp.int32),
            pltpu.VMEM((128,), _f32),
            pltpu.VMEM((128,), _f32),
            pltpu.VMEM_SHARED((NPAD,), _f32),
        ],
    )
    return f(dst_deg)


def _prep_body(deg_ref, h_ref, c1_ref, g0_ref, c2_ref, invx_ref):
    i = pl.program_id(0)
    rb = deg_ref.shape[0]
    rows = i * rb + lax.broadcasted_iota(jnp.int32, (rb, 1), 0)
    valid = rows < N
    d = jnp.maximum(deg_ref[...], 1.0)
    norm = jnp.where(valid, lax.rsqrt(d), 0.0)
    inv = jnp.where(valid, jnp.sqrt(d), 0.0)
    h = h_ref[...]
    g0lo = norm * h[:, :HALF]
    g0hi = norm * h[:, HALF:]
    g0_ref[0] = g0lo
    g0_ref[1] = g0hi
    c2_ref[0] = ALPHA * g0lo
    c2_ref[1] = ALPHA * g0hi
    ones = jnp.ones((1, HALF), _f32)
    c1_ref[...] = ((1.0 - ALPHA) * norm * norm) * ones
    invx_ref[0] = inv * ones
    invx_ref[1] = inv * ones


def _prep(deg_col, h_pad):
    RB = 2560
    grid = (NPAD // RB,)
    return pl.pallas_call(
        _prep_body,
        grid=grid,
        in_specs=[
            pl.BlockSpec((RB, 1), lambda i: (i, 0)),
            pl.BlockSpec((RB, OUT), lambda i: (i, 0)),
        ],
        out_specs=[
            pl.BlockSpec((RB, HALF), lambda i: (i, 0)),
            pl.BlockSpec((2, RB, HALF), lambda i: (0, i, 0)),
            pl.BlockSpec((2, RB, HALF), lambda i: (0, i, 0)),
            pl.BlockSpec((2, RB, HALF), lambda i: (0, i, 0)),
        ],
        out_shape=[
            jax.ShapeDtypeStruct((NPAD, HALF), _f32),
            jax.ShapeDtypeStruct((2, NPAD, HALF), _f32),
            jax.ShapeDtypeStruct((2, NPAD, HALF), _f32),
            jax.ShapeDtypeStruct((2, NPAD, HALF), _f32),
        ],
    )(deg_col, h_pad)


CHUNK = 128
CCH = EPT // CHUNK
BLK = 20
NBLK = CCH // BLK
NSLOT = 5
EW = RT // 2


def _prop_body(src_hbm, dst_hbm, c1_hbm, c2_hbm, g0_hbm, gk_out,
               sidx, didx, rows, c1_b, c2_b, nbuf, zbuf, gs, ss, g_sh, agg_sh):
    c = lax.axis_index("core")
    s = lax.axis_index("subcore")
    rowbase = s * RT
    hrowbase = c * NPAD + rowbase
    chunkbase = s * CCH

    pltpu.sync_copy(c1_hbm.at[pl.ds(rowbase, RT)], c1_b)
    pltpu.sync_copy(c2_hbm.at[pl.ds(hrowbase, RT)], c2_b)
    pltpu.sync_copy(g0_hbm.at[pl.ds(hrowbase, RT)], g_sh.at[pl.ds(rowbase, RT)])

    @pl.loop(0, 64)
    def _(i):
        zbuf[i, pl.ds(0, 16)] = jnp.zeros((16,), _f32)
        zbuf[i, pl.ds(16, 16)] = jnp.zeros((16,), _f32)

    @pl.loop(0, 10)
    def _(z):
        pltpu.sync_copy(zbuf, agg_sh.at[pl.ds(rowbase + z * 64, 64)])

    plsc.subcore_barrier()

    def wait_gather(j, m):
        pltpu.make_async_copy(g_sh.at[sidx.at[j]], rows.at[m],
                              gs.at[m]).wait()

    def wait_scatter(j, m):
        pltpu.make_async_copy(rows.at[m], agg_sh.at[didx.at[j]],
                              ss.at[m]).wait()

    def fire_gather(j, m):
        pltpu.async_copy(g_sh.at[sidx.at[j]], rows.at[m], gs.at[m])

    def fire_scatter(j, m):
        pltpu.async_copy(rows.at[m], agg_sh.at[didx.at[j]], ss.at[m],
                         add=True)

    @pl.loop(0, K)
    def _(t):
        @pl.loop(0, NBLK)
        def _(b):
            pltpu.sync_copy(src_hbm.at[pl.ds(chunkbase + b * BLK, BLK)], sidx)
            pltpu.sync_copy(dst_hbm.at[pl.ds(chunkbase + b * BLK, BLK)], didx)

            @pl.loop(0, BLK, step=NSLOT)
            def _(j0):
                for cc in range(NSLOT):
                    j = j0 + cc
                    m2 = (cc + 2) % NSLOT

                    @pl.when(j >= NSLOT)
                    def _():
                        wait_scatter(j - NSLOT, cc)

                    fire_gather(j, cc)

                    @pl.when(j >= 3)
                    def _():
                        wait_gather(j - 3, m2)
                        fire_scatter(j - 3, m2)

            for j in (BLK - 3, BLK - 2, BLK - 1):
                wait_gather(j, j % NSLOT)
                fire_scatter(j, j % NSLOT)
            for j in range(BLK - NSLOT, BLK):
                wait_scatter(j, j % NSLOT)

        plsc.subcore_barrier()

        for half in (0, 1):
            hb = half * EW
            pltpu.sync_copy(agg_sh.at[pl.ds(rowbase + hb, EW)], nbuf)

            @pl.loop(0, EW, unroll=8)
            def _(i):
                lo = pl.ds(0, 16)
                hi = pl.ds(16, 16)
                nbuf[i, lo] = c1_b[hb + i, lo] * nbuf[i, lo] + c2_b[hb + i, lo]
                nbuf[i, hi] = c1_b[hb + i, hi] * nbuf[i, hi] + c2_b[hb + i, hi]

            pltpu.sync_copy(nbuf, g_sh.at[pl.ds(rowbase + hb, EW)])

        @pl.loop(0, 10)
        def _(z):
            pltpu.sync_copy(zbuf, agg_sh.at[pl.ds(rowbase + z * 64, 64)])

        plsc.subcore_barrier()

    pltpu.sync_copy(g_sh.at[pl.ds(rowbase, RT)], gk_out.at[pl.ds(hrowbase, RT)])


def _propagate(src_idx, dst_idx, c1x, c2f, g0f):
    mesh = plsc.VectorSubcoreMesh(core_axis_name="core",
                                  subcore_axis_name="subcore")
    f = pl.kernel(
        _prop_body,
        out_type=jax.ShapeDtypeStruct((2 * NPAD, HALF), _f32),
        mesh=mesh,
        compiler_params=pltpu.CompilerParams(use_tc_tiling_on_sc=False),
        scratch_types=[
            pltpu.VMEM((BLK, CHUNK), jnp.int32),
            pltpu.VMEM((BLK, CHUNK), jnp.int32),
            pltpu.VMEM((NSLOT, CHUNK, HALF), _f32),
            pltpu.VMEM((RT, HALF), _f32),
            pltpu.VMEM((RT, HALF), _f32),
            pltpu.VMEM((EW, HALF), _f32),
            pltpu.VMEM((64, HALF), _f32),
            pltpu.SemaphoreType.DMA((NSLOT,)),
            pltpu.SemaphoreType.DMA((NSLOT,)),
            pltpu.VMEM_SHARED((NPAD, HALF), _f32),
            pltpu.VMEM_SHARED((NPAD, HALF), _f32),
        ],
    )
    return f(src_idx, dst_idx, c1x, c2f, g0f)


def _final_body(gk_ref, invx_ref, h_ref):
    h_ref[...] = jnp.concatenate(
        [gk_ref[0] * invx_ref[0], gk_ref[1] * invx_ref[1]], axis=1)


def _final(gk, invx):
    RB = 2000
    grid = (N // RB,)
    return pl.pallas_call(
        _final_body,
        grid=grid,
        in_specs=[
            pl.BlockSpec((2, RB, HALF), lambda i: (0, i, 0)),
            pl.BlockSpec((2, RB, HALF), lambda i: (0, i, 0)),
        ],
        out_specs=pl.BlockSpec((RB, OUT), lambda i: (i, 0)),
        out_shape=jax.ShapeDtypeStruct((N, OUT), _f32),
    )(gk, invx)


@jax.jit
def kernel(feats, edge_index, W1, b1, W2, b2):
    src = edge_index[0]
    dst = edge_index[1]
    pad = E_PAD - E
    src_p = jnp.concatenate([src, jnp.full((pad,), N, jnp.int32)])
    dst_p = jnp.concatenate([dst, jnp.full((pad,), N, jnp.int32)])
    src_idx = src_p.reshape(16 * CCH, CHUNK)
    dst_idx = dst_p.reshape(16 * CCH, CHUNK)
    dst_deg = dst_p.reshape(32, DEG_CH, 128)

    h1, h = _mlp(feats, W1, b1, W2, b2)
    deg2 = _degree(dst_deg)
    deg_col = (deg2[0] + deg2[1]).reshape(NPAD, 1)
    h_pad = jnp.pad(h, ((0, NPAD - N), (0, 0)))
    c1x, g0, c2, invx = _prep(deg_col, h_pad)
    gk = _propagate(src_idx, dst_idx, c1x,
                    c2.reshape(2 * NPAD, HALF), g0.reshape(2 * NPAD, HALF))
    hout = _final(gk.reshape(2, NPAD, HALF), invx)
    return (h1, hout)


# speedup vs baseline: 1.9830x; 1.0645x over previous
import functools

import jax
import jax.numpy as jnp
from jax import lax
from jax.experimental import pallas as pl
from jax.experimental.pallas import tpu as pltpu
from jax.experimental.pallas import tpu_sc as plsc

N = 10000
E = 320000
IN_DIM = 128
HID = 128
OUT = 64
HALF = 32
K = 10
ALPHA = 0.1

NPAD = 10240
RT = NPAD // 16
CH = 160
EPT = CH * 128
E_PAD = 16 * EPT
DEG_CH = E_PAD // (32 * 128)

_f32 = jnp.float32


def _mlp_body(feats_ref, w1_ref, b1_ref, w2_ref, b2_ref, h1_ref, h_ref):
    a = jnp.dot(feats_ref[...], w1_ref[...],
                preferred_element_type=_f32) + b1_ref[...]
    h1_ref[...] = a
    h_ref[...] = jnp.dot(jnp.maximum(a, 0.0), w2_ref[...],
                         preferred_element_type=_f32) + b2_ref[...]


def _mlp(feats, W1, b1, W2, b2):
    RB = 2000
    grid = (N // RB,)
    return pl.pallas_call(
        _mlp_body,
        grid=grid,
        in_specs=[
            pl.BlockSpec((RB, IN_DIM), lambda i: (i, 0)),
            pl.BlockSpec((IN_DIM, HID), lambda i: (0, 0)),
            pl.BlockSpec((1, HID), lambda i: (0, 0)),
            pl.BlockSpec((HID, OUT), lambda i: (0, 0)),
            pl.BlockSpec((1, OUT), lambda i: (0, 0)),
        ],
        out_specs=[
            pl.BlockSpec((RB, HID), lambda i: (i, 0)),
            pl.BlockSpec((RB, OUT), lambda i: (i, 0)),
        ],
        out_shape=[
            jax.ShapeDtypeStruct((N, HID), _f32),
            jax.ShapeDtypeStruct((N, OUT), _f32),
        ],
    )(feats, W1.astype(_f32), b1.reshape(1, HID), W2.astype(_f32),
      b2.reshape(1, OUT))


def _deg_body(dst_hbm, deg_out, idx_v, ones_v, zb_v, deg_sh):
    c = lax.axis_index("core")
    s = lax.axis_index("subcore")
    wid = c * 16 + s

    @pl.loop(0, 8)
    def _(i):
        ones_v[pl.ds(i * 16, 16)] = jnp.full((16,), 1.0, _f32)
        zb_v[pl.ds(i * 16, 16)] = jnp.zeros((16,), _f32)

    @pl.loop(0, 5)
    def _(z):
        pltpu.sync_copy(zb_v, deg_sh.at[pl.ds(s * RT + z * 128, 128)])

    pltpu.sync_copy(dst_hbm.at[wid], idx_v)
    plsc.subcore_barrier()

    @pl.loop(0, DEG_CH)
    def _(j):
        pltpu.sync_copy(ones_v, deg_sh.at[idx_v.at[j]], add=True)

    plsc.subcore_barrier()

    @pl.when(s == 0)
    def _():
        pltpu.sync_copy(deg_sh, deg_out.at[c])


def _degree(dst_deg):
    mesh = plsc.VectorSubcoreMesh(core_axis_name="core",
                                  subcore_axis_name="subcore")
    f = pl.kernel(
        _deg_body,
        out_type=jax.ShapeDtypeStruct((2, NPAD), _f32),
        mesh=mesh,
        scratch_types=[
            pltpu.VMEM((DEG_CH, 128), jnp.int32),
            pltpu.VMEM((128,), _f32),
            pltpu.VMEM((128,), _f32),
            pltpu.VMEM_SHARED((NPAD,), _f32),
        ],
    )
    return f(dst_deg)


def _prep_body(deg_ref, h_ref, c1_ref, g0_ref, c2_ref, invx_ref):
    i = pl.program_id(0)
    rb = deg_ref.shape[0]
    rows = i * rb + lax.broadcasted_iota(jnp.int32, (rb, 1), 0)
    valid = rows < N
    d = jnp.maximum(deg_ref[...], 1.0)
    norm = jnp.where(valid, lax.rsqrt(d), 0.0)
    inv = jnp.where(valid, jnp.sqrt(d), 0.0)
    h = h_ref[...]
    g0lo = norm * h[:, :HALF]
    g0hi = norm * h[:, HALF:]
    g0_ref[0] = g0lo
    g0_ref[1] = g0hi
    c2_ref[0] = ALPHA * g0lo
    c2_ref[1] = ALPHA * g0hi
    ones = jnp.ones((1, HALF), _f32)
    c1_ref[...] = ((1.0 - ALPHA) * norm * norm) * ones
    invx_ref[0] = inv * ones
    invx_ref[1] = inv * ones


def _prep(deg_col, h_pad):
    RB = 2560
    grid = (NPAD // RB,)
    return pl.pallas_call(
        _prep_body,
        grid=grid,
        in_specs=[
            pl.BlockSpec((RB, 1), lambda i: (i, 0)),
            pl.BlockSpec((RB, OUT), lambda i: (i, 0)),
        ],
        out_specs=[
            pl.BlockSpec((RB, HALF), lambda i: (i, 0)),
            pl.BlockSpec((2, RB, HALF), lambda i: (0, i, 0)),
            pl.BlockSpec((2, RB, HALF), lambda i: (0, i, 0)),
            pl.BlockSpec((2, RB, HALF), lambda i: (0, i, 0)),
        ],
        out_shape=[
            jax.ShapeDtypeStruct((NPAD, HALF), _f32),
            jax.ShapeDtypeStruct((2, NPAD, HALF), _f32),
            jax.ShapeDtypeStruct((2, NPAD, HALF), _f32),
            jax.ShapeDtypeStruct((2, NPAD, HALF), _f32),
        ],
    )(deg_col, h_pad)


CHUNK = 128
CCH = EPT // CHUNK
BLK = 40
NBLK = CCH // BLK
NSLOT = 4
STAG = 2
EW = RT // 2


def _prop_body(src_hbm, dst_hbm, c1_hbm, c2_hbm, g0_hbm, gk_out,
               sidx, didx, rows, c1_b, c2_b, nbuf, zbuf, gs, ss, g_sh, agg_sh):
    c = lax.axis_index("core")
    s = lax.axis_index("subcore")
    rowbase = s * RT
    hrowbase = c * NPAD + rowbase
    chunkbase = s * CCH

    pltpu.sync_copy(c1_hbm.at[pl.ds(rowbase, RT)], c1_b)
    pltpu.sync_copy(c2_hbm.at[pl.ds(hrowbase, RT)], c2_b)
    pltpu.sync_copy(g0_hbm.at[pl.ds(hrowbase, RT)], g_sh.at[pl.ds(rowbase, RT)])

    @pl.loop(0, 32)
    def _(i):
        zbuf[i, pl.ds(0, 16)] = jnp.zeros((16,), _f32)
        zbuf[i, pl.ds(16, 16)] = jnp.zeros((16,), _f32)

    @pl.loop(0, 20)
    def _(z):
        pltpu.sync_copy(zbuf, agg_sh.at[pl.ds(rowbase + z * 32, 32)])

    plsc.subcore_barrier()

    def wait_gather(j, m):
        pltpu.make_async_copy(g_sh.at[sidx.at[j]], rows.at[m],
                              gs.at[m]).wait()

    def wait_scatter(j, m):
        pltpu.make_async_copy(rows.at[m], agg_sh.at[didx.at[j]],
                              ss.at[m]).wait()

    def fire_gather(j, m):
        pltpu.async_copy(g_sh.at[sidx.at[j]], rows.at[m], gs.at[m])

    def fire_scatter(j, m):
        pltpu.async_copy(rows.at[m], agg_sh.at[didx.at[j]], ss.at[m],
                         add=True)

    @pl.loop(0, K)
    def _(t):
        @pl.loop(0, NBLK)
        def _(b):
            pltpu.sync_copy(src_hbm.at[pl.ds(chunkbase + b * BLK, BLK)], sidx)
            pltpu.sync_copy(dst_hbm.at[pl.ds(chunkbase + b * BLK, BLK)], didx)

            @pl.loop(0, BLK, step=NSLOT)
            def _(j0):
                for cc in range(NSLOT):
                    j = j0 + cc
                    mS = (cc + NSLOT - STAG) % NSLOT

                    @pl.when(j >= NSLOT)
                    def _():
                        wait_scatter(j - NSLOT, cc)

                    fire_gather(j, cc)

                    @pl.when(j >= STAG)
                    def _():
                        wait_gather(j - STAG, mS)
                        fire_scatter(j - STAG, mS)

            for j in range(BLK - STAG, BLK):
                wait_gather(j, j % NSLOT)
                fire_scatter(j, j % NSLOT)
            for j in range(BLK - NSLOT, BLK):
                wait_scatter(j, j % NSLOT)

        plsc.subcore_barrier()

        for half in (0, 1):
            hb = half * EW
            pltpu.sync_copy(agg_sh.at[pl.ds(rowbase + hb, EW)], nbuf)

            @pl.loop(0, EW, unroll=8)
            def _(i):
                lo = pl.ds(0, 16)
                hi = pl.ds(16, 16)
                nbuf[i, lo] = c1_b[hb + i, lo] * nbuf[i, lo] + c2_b[hb + i, lo]
                nbuf[i, hi] = c1_b[hb + i, hi] * nbuf[i, hi] + c2_b[hb + i, hi]

            pltpu.sync_copy(nbuf, g_sh.at[pl.ds(rowbase + hb, EW)])

        @pl.loop(0, 20)
        def _(z):
            pltpu.sync_copy(zbuf, agg_sh.at[pl.ds(rowbase + z * 32, 32)])

        plsc.subcore_barrier()

    pltpu.sync_copy(g_sh.at[pl.ds(rowbase, RT)], gk_out.at[pl.ds(hrowbase, RT)])


def _propagate(src_idx, dst_idx, c1x, c2f, g0f):
    mesh = plsc.VectorSubcoreMesh(core_axis_name="core",
                                  subcore_axis_name="subcore")
    f = pl.kernel(
        _prop_body,
        out_type=jax.ShapeDtypeStruct((2 * NPAD, HALF), _f32),
        mesh=mesh,
        compiler_params=pltpu.CompilerParams(use_tc_tiling_on_sc=False),
        scratch_types=[
            pltpu.VMEM((BLK, CHUNK), jnp.int32),
            pltpu.VMEM((BLK, CHUNK), jnp.int32),
            pltpu.VMEM((NSLOT, CHUNK, HALF), _f32),
            pltpu.VMEM((RT, HALF), _f32),
            pltpu.VMEM((RT, HALF), _f32),
            pltpu.VMEM((EW, HALF), _f32),
            pltpu.VMEM((32, HALF), _f32),
            pltpu.SemaphoreType.DMA((NSLOT,)),
            pltpu.SemaphoreType.DMA((NSLOT,)),
            pltpu.VMEM_SHARED((NPAD, HALF), _f32),
            pltpu.VMEM_SHARED((NPAD, HALF), _f32),
        ],
    )
    return f(src_idx, dst_idx, c1x, c2f, g0f)


def _final_body(gk_ref, invx_ref, h_ref):
    h_ref[...] = jnp.concatenate(
        [gk_ref[0] * invx_ref[0], gk_ref[1] * invx_ref[1]], axis=1)


def _final(gk, invx):
    RB = 2000
    grid = (N // RB,)
    return pl.pallas_call(
        _final_body,
        grid=grid,
        in_specs=[
            pl.BlockSpec((2, RB, HALF), lambda i: (0, i, 0)),
            pl.BlockSpec((2, RB, HALF), lambda i: (0, i, 0)),
        ],
        out_specs=pl.BlockSpec((RB, OUT), lambda i: (i, 0)),
        out_shape=jax.ShapeDtypeStruct((N, OUT), _f32),
    )(gk, invx)


@jax.jit
def kernel(feats, edge_index, W1, b1, W2, b2):
    src = edge_index[0]
    dst = edge_index[1]
    pad = E_PAD - E
    src_p = jnp.concatenate([src, jnp.full((pad,), N, jnp.int32)])
    dst_p = jnp.concatenate([dst, jnp.full((pad,), N, jnp.int32)])
    src_idx = src_p.reshape(16 * CCH, CHUNK)
    dst_idx = dst_p.reshape(16 * CCH, CHUNK)
    dst_deg = dst_p.reshape(32, DEG_CH, 128)

    h1, h = _mlp(feats, W1, b1, W2, b2)
    deg2 = _degree(dst_deg)
    deg_col = (deg2[0] + deg2[1]).reshape(NPAD, 1)
    h_pad = jnp.pad(h, ((0, NPAD - N), (0, 0)))
    c1x, g0, c2, invx = _prep(deg_col, h_pad)
    gk = _propagate(src_idx, dst_idx, c1x,
                    c2.reshape(2 * NPAD, HALF), g0.reshape(2 * NPAD, HALF))
    hout = _final(gk.reshape(2, NPAD, HALF), invx)
    return (h1, hout)


# speedup vs baseline: 1.9883x; 1.0027x over previous
import functools

import jax
import jax.numpy as jnp
from jax import lax
from jax.experimental import pallas as pl
from jax.experimental.pallas import tpu as pltpu
from jax.experimental.pallas import tpu_sc as plsc

N = 10000
E = 320000
IN_DIM = 128
HID = 128
OUT = 64
HALF = 32
K = 10
ALPHA = 0.1

NPAD = 10240
RT = NPAD // 16
CH = 160
EPT = CH * 128
E_PAD = 16 * EPT
DEG_CH = E_PAD // (32 * 128)

_f32 = jnp.float32


def _mlp_body(feats_ref, w1_ref, b1_ref, w2_ref, b2_ref, h1_ref, h_ref):
    a = jnp.dot(feats_ref[...], w1_ref[...],
                preferred_element_type=_f32) + b1_ref[...]
    h1_ref[...] = a
    h_ref[...] = jnp.dot(jnp.maximum(a, 0.0), w2_ref[...],
                         preferred_element_type=_f32) + b2_ref[...]


def _mlp(feats, W1, b1, W2, b2):
    RB = 2000
    grid = (N // RB,)
    return pl.pallas_call(
        _mlp_body,
        grid=grid,
        in_specs=[
            pl.BlockSpec((RB, IN_DIM), lambda i: (i, 0)),
            pl.BlockSpec((IN_DIM, HID), lambda i: (0, 0)),
            pl.BlockSpec((1, HID), lambda i: (0, 0)),
            pl.BlockSpec((HID, OUT), lambda i: (0, 0)),
            pl.BlockSpec((1, OUT), lambda i: (0, 0)),
        ],
        out_specs=[
            pl.BlockSpec((RB, HID), lambda i: (i, 0)),
            pl.BlockSpec((RB, OUT), lambda i: (i, 0)),
        ],
        out_shape=[
            jax.ShapeDtypeStruct((N, HID), _f32),
            jax.ShapeDtypeStruct((N, OUT), _f32),
        ],
    )(feats, W1.astype(_f32), b1.reshape(1, HID), W2.astype(_f32),
      b2.reshape(1, OUT))


def _deg_body(dst_hbm, deg_out, idx_v, ones_v, zb_v, deg_sh):
    c = lax.axis_index("core")
    s = lax.axis_index("subcore")
    wid = c * 16 + s

    @pl.loop(0, 8)
    def _(i):
        ones_v[pl.ds(i * 16, 16)] = jnp.full((16,), 1.0, _f32)
        zb_v[pl.ds(i * 16, 16)] = jnp.zeros((16,), _f32)

    @pl.loop(0, 5)
    def _(z):
        pltpu.sync_copy(zb_v, deg_sh.at[pl.ds(s * RT + z * 128, 128)])

    pltpu.sync_copy(dst_hbm.at[wid], idx_v)
    plsc.subcore_barrier()

    @pl.loop(0, DEG_CH)
    def _(j):
        pltpu.sync_copy(ones_v, deg_sh.at[idx_v.at[j]], add=True)

    plsc.subcore_barrier()

    @pl.when(s == 0)
    def _():
        pltpu.sync_copy(deg_sh, deg_out.at[c])


def _degree(dst_deg):
    mesh = plsc.VectorSubcoreMesh(core_axis_name="core",
                                  subcore_axis_name="subcore")
    f = pl.kernel(
        _deg_body,
        out_type=jax.ShapeDtypeStruct((2, NPAD), _f32),
        mesh=mesh,
        scratch_types=[
            pltpu.VMEM((DEG_CH, 128), jnp.int32),
            pltpu.VMEM((128,), _f32),
            pltpu.VMEM((128,), _f32),
            pltpu.VMEM_SHARED((NPAD,), _f32),
        ],
    )
    return f(dst_deg)


def _prep_body(deg_ref, h_ref, c1_ref, g0_ref, c2_ref, invx_ref):
    i = pl.program_id(0)
    rb = deg_ref.shape[0]
    rows = i * rb + lax.broadcasted_iota(jnp.int32, (rb, 1), 0)
    valid = rows < N
    d = jnp.maximum(deg_ref[...], 1.0)
    norm = jnp.where(valid, lax.rsqrt(d), 0.0)
    inv = jnp.where(valid, jnp.sqrt(d), 0.0)
    h = h_ref[...]
    g0lo = norm * h[:, :HALF]
    g0hi = norm * h[:, HALF:]
    g0_ref[0] = g0lo
    g0_ref[1] = g0hi
    c2_ref[0] = ALPHA * g0lo
    c2_ref[1] = ALPHA * g0hi
    ones = jnp.ones((1, HALF), _f32)
    c1_ref[...] = ((1.0 - ALPHA) * norm * norm) * ones
    invx_ref[0] = inv * ones
    invx_ref[1] = inv * ones


def _prep(deg_col, h_pad):
    RB = 2560
    grid = (NPAD // RB,)
    return pl.pallas_call(
        _prep_body,
        grid=grid,
        in_specs=[
            pl.BlockSpec((RB, 1), lambda i: (i, 0)),
            pl.BlockSpec((RB, OUT), lambda i: (i, 0)),
        ],
        out_specs=[
            pl.BlockSpec((RB, HALF), lambda i: (i, 0)),
            pl.BlockSpec((2, RB, HALF), lambda i: (0, i, 0)),
            pl.BlockSpec((2, RB, HALF), lambda i: (0, i, 0)),
            pl.BlockSpec((2, RB, HALF), lambda i: (0, i, 0)),
        ],
        out_shape=[
            jax.ShapeDtypeStruct((NPAD, HALF), _f32),
            jax.ShapeDtypeStruct((2, NPAD, HALF), _f32),
            jax.ShapeDtypeStruct((2, NPAD, HALF), _f32),
            jax.ShapeDtypeStruct((2, NPAD, HALF), _f32),
        ],
    )(deg_col, h_pad)


CHUNK = 64
CCH = EPT // CHUNK
BLK = 80
NBLK = CCH // BLK
NSLOT = 8
STAG = 4
EW = RT // 2


def _prop_body(src_hbm, dst_hbm, c1_hbm, c2_hbm, g0_hbm, gk_out,
               sidx, didx, rows, c1_b, c2_b, nbuf, zbuf, gs, ss, g_sh, agg_sh):
    c = lax.axis_index("core")
    s = lax.axis_index("subcore")
    rowbase = s * RT
    hrowbase = c * NPAD + rowbase
    chunkbase = s * CCH

    pltpu.sync_copy(c1_hbm.at[pl.ds(rowbase, RT)], c1_b)
    pltpu.sync_copy(c2_hbm.at[pl.ds(hrowbase, RT)], c2_b)
    pltpu.sync_copy(g0_hbm.at[pl.ds(hrowbase, RT)], g_sh.at[pl.ds(rowbase, RT)])

    @pl.loop(0, 32)
    def _(i):
        zbuf[i, pl.ds(0, 16)] = jnp.zeros((16,), _f32)
        zbuf[i, pl.ds(16, 16)] = jnp.zeros((16,), _f32)

    @pl.loop(0, 20)
    def _(z):
        pltpu.sync_copy(zbuf, agg_sh.at[pl.ds(rowbase + z * 32, 32)])

    plsc.subcore_barrier()

    def wait_gather(j, m):
        pltpu.make_async_copy(g_sh.at[sidx.at[j]], rows.at[m],
                              gs.at[m]).wait()

    def wait_scatter(j, m):
        pltpu.make_async_copy(rows.at[m], agg_sh.at[didx.at[j]],
                              ss.at[m]).wait()

    def fire_gather(j, m):
        pltpu.async_copy(g_sh.at[sidx.at[j]], rows.at[m], gs.at[m])

    def fire_scatter(j, m):
        pltpu.async_copy(rows.at[m], agg_sh.at[didx.at[j]], ss.at[m],
                         add=True)

    @pl.loop(0, K)
    def _(t):
        @pl.loop(0, NBLK)
        def _(b):
            pltpu.sync_copy(src_hbm.at[pl.ds(chunkbase + b * BLK, BLK)], sidx)
            pltpu.sync_copy(dst_hbm.at[pl.ds(chunkbase + b * BLK, BLK)], didx)

            @pl.loop(0, BLK, step=NSLOT)
            def _(j0):
                for cc in range(NSLOT):
                    j = j0 + cc
                    mS = (cc + NSLOT - STAG) % NSLOT

                    @pl.when(j >= NSLOT)
                    def _():
                        wait_scatter(j - NSLOT, cc)

                    fire_gather(j, cc)

                    @pl.when(j >= STAG)
                    def _():
                        wait_gather(j - STAG, mS)
                        fire_scatter(j - STAG, mS)

            for j in range(BLK - STAG, BLK):
                wait_gather(j, j % NSLOT)
                fire_scatter(j, j % NSLOT)
            for j in range(BLK - NSLOT, BLK):
                wait_scatter(j, j % NSLOT)

        plsc.subcore_barrier()

        for half in (0, 1):
            hb = half * EW
            pltpu.sync_copy(agg_sh.at[pl.ds(rowbase + hb, EW)], nbuf)

            @pl.loop(0, EW, unroll=8)
            def _(i):
                lo = pl.ds(0, 16)
                hi = pl.ds(16, 16)
                nbuf[i, lo] = c1_b[hb + i, lo] * nbuf[i, lo] + c2_b[hb + i, lo]
                nbuf[i, hi] = c1_b[hb + i, hi] * nbuf[i, hi] + c2_b[hb + i, hi]

            pltpu.sync_copy(nbuf, g_sh.at[pl.ds(rowbase + hb, EW)])

        @pl.loop(0, 20)
        def _(z):
            pltpu.sync_copy(zbuf, agg_sh.at[pl.ds(rowbase + z * 32, 32)])

        plsc.subcore_barrier()

    pltpu.sync_copy(g_sh.at[pl.ds(rowbase, RT)], gk_out.at[pl.ds(hrowbase, RT)])


def _propagate(src_idx, dst_idx, c1x, c2f, g0f):
    mesh = plsc.VectorSubcoreMesh(core_axis_name="core",
                                  subcore_axis_name="subcore")
    f = pl.kernel(
        _prop_body,
        out_type=jax.ShapeDtypeStruct((2 * NPAD, HALF), _f32),
        mesh=mesh,
        compiler_params=pltpu.CompilerParams(use_tc_tiling_on_sc=False),
        scratch_types=[
            pltpu.VMEM((BLK, CHUNK), jnp.int32),
            pltpu.VMEM((BLK, CHUNK), jnp.int32),
            pltpu.VMEM((NSLOT, CHUNK, HALF), _f32),
            pltpu.VMEM((RT, HALF), _f32),
            pltpu.VMEM((RT, HALF), _f32),
            pltpu.VMEM((EW, HALF), _f32),
            pltpu.VMEM((32, HALF), _f32),
            pltpu.SemaphoreType.DMA((NSLOT,)),
            pltpu.SemaphoreType.DMA((NSLOT,)),
            pltpu.VMEM_SHARED((NPAD, HALF), _f32),
            pltpu.VMEM_SHARED((NPAD, HALF), _f32),
        ],
    )
    return f(src_idx, dst_idx, c1x, c2f, g0f)


def _final_body(gk_ref, invx_ref, h_ref):
    h_ref[...] = jnp.concatenate(
        [gk_ref[0] * invx_ref[0], gk_ref[1] * invx_ref[1]], axis=1)


def _final(gk, invx):
    RB = 2000
    grid = (N // RB,)
    return pl.pallas_call(
        _final_body,
        grid=grid,
        in_specs=[
            pl.BlockSpec((2, RB, HALF), lambda i: (0, i, 0)),
            pl.BlockSpec((2, RB, HALF), lambda i: (0, i, 0)),
        ],
        out_specs=pl.BlockSpec((RB, OUT), lambda i: (i, 0)),
        out_shape=jax.ShapeDtypeStruct((N, OUT), _f32),
    )(gk, invx)


@jax.jit
def kernel(feats, edge_index, W1, b1, W2, b2):
    src = edge_index[0]
    dst = edge_index[1]
    pad = E_PAD - E
    src_p = jnp.concatenate([src, jnp.full((pad,), N, jnp.int32)])
    dst_p = jnp.concatenate([dst, jnp.full((pad,), N, jnp.int32)])
    src_idx = src_p.reshape(16 * CCH, CHUNK)
    dst_idx = dst_p.reshape(16 * CCH, CHUNK)
    dst_deg = dst_p.reshape(32, DEG_CH, 128)

    h1, h = _mlp(feats, W1, b1, W2, b2)
    deg2 = _degree(dst_deg)
    deg_col = (deg2[0] + deg2[1]).reshape(NPAD, 1)
    h_pad = jnp.pad(h, ((0, NPAD - N), (0, 0)))
    c1x, g0, c2, invx = _prep(deg_col, h_pad)
    gk = _propagate(src_idx, dst_idx, c1x,
                    c2.reshape(2 * NPAD, HALF), g0.reshape(2 * NPAD, HALF))
    hout = _final(gk.reshape(2, NPAD, HALF), invx)
    return (h1, hout)
